# Initial kernel scaffold; baseline (speedup 1.0000x reference)
import jax
import jax.numpy as jnp
from jax.experimental import pallas as pl


def kernel(edge_index, edge_weight, emb_weight, W1, b1, W2, b2):
    raise NotImplementedError("write your pallas kernel here")


# speedup vs baseline: 3.1846x; 3.1846x over previous
import functools

import jax
import jax.numpy as jnp
from jax import lax
from jax.experimental import pallas as pl
from jax.experimental.pallas import tpu as pltpu
from jax.experimental.pallas import tpu_sc as plsc

N = 50000
E = 800000
HALF = 32
NCORE = 2
NSUB = 16
CHUNK = 128
NCHUNK = 392
EPS = CHUNK * NCHUNK
E_PAD = EPS * NSUB
STRIPE = 3128
NP = STRIPE * NSUB


def _spmm_sc(srcp, dstp, wp, xflat, zrows):
    mesh = plsc.VectorSubcoreMesh(
        core_axis_name="c", subcore_axis_name="s", num_cores=NCORE, num_subcores=NSUB
    )

    @functools.partial(
        pl.kernel,
        out_type=jax.ShapeDtypeStruct((NCORE * NP, HALF), jnp.float32),
        mesh=mesh,
        scratch_types=[
            pltpu.VMEM((CHUNK,), jnp.int32),
            pltpu.VMEM((CHUNK,), jnp.int32),
            pltpu.VMEM((CHUNK,), jnp.float32),
            pltpu.VMEM((CHUNK, HALF), jnp.float32),
            pltpu.VMEM_SHARED((NP, HALF), jnp.float32),
            pltpu.SemaphoreType.DMA,
        ],
        compiler_params=pltpu.CompilerParams(use_tc_tiling_on_sc=False),
    )
    def k(src_hbm, dst_hbm, w_hbm, x_hbm, z_hbm, y_hbm, src_v, dst_v, w_v, rows, acc, sem):
        c = lax.axis_index("c")
        s = lax.axis_index("s")
        cbase = c * NP
        pltpu.sync_copy(z_hbm.at[pl.ds(s * STRIPE, STRIPE)], acc.at[pl.ds(s * STRIPE, STRIPE)])
        plsc.subcore_barrier()

        ebase = s * EPS

        @pl.loop(0, NCHUNK)
        def _(j):
            base = ebase + j * CHUNK
            pltpu.sync_copy(src_hbm.at[pl.ds(base, CHUNK)], src_v)
            pltpu.sync_copy(dst_hbm.at[pl.ds(base, CHUNK)], dst_v)
            pltpu.sync_copy(w_hbm.at[pl.ds(base, CHUNK)], w_v)

            @pl.loop(0, CHUNK // 16)
            def _(i):
                sl = pl.ds(i * 16, 16)
                src_v[sl] = src_v[sl] + cbase

            pltpu.async_copy(x_hbm.at[src_v], rows, sem).wait()

            @pl.loop(0, CHUNK // 16)
            def _(g):
                w16 = w_v[pl.ds(g * 16, 16)]
                for kk in range(16):
                    r = g * 16 + kk
                    wr = w16[kk]
                    rows[r, pl.ds(0, 16)] = rows[r, pl.ds(0, 16)] * wr
                    rows[r, pl.ds(16, 16)] = rows[r, pl.ds(16, 16)] * wr

            pltpu.sync_copy(rows, acc.at[dst_v], add=True)

        plsc.subcore_barrier()
        pltpu.sync_copy(
            acc.at[pl.ds(s * STRIPE, STRIPE)],
            y_hbm.at[pl.ds(cbase + s * STRIPE, STRIPE)],
        )

    return k(srcp, dstp, wp, xflat, zrows)


_DB = 3128
_DG = NP // _DB


def _dense_body(split_out, y_ref, wa_ref, wb_ref, b_ref, o_ref):
    h = (
        jnp.dot(y_ref[0], wa_ref[...], preferred_element_type=jnp.float32,
                precision=lax.Precision.HIGHEST)
        + jnp.dot(y_ref[1], wb_ref[...], preferred_element_type=jnp.float32,
                  precision=lax.Precision.HIGHEST)
        + b_ref[...]
    )
    h = jnp.maximum(h, 0.0)
    if split_out:
        o_ref[0] = h[:, 0:HALF]
        o_ref[1] = h[:, HALF : 2 * HALF]
    else:
        o_ref[...] = h


def _dense_halves_tc(yflat, wa, wb, b):
    y3 = yflat.reshape(NCORE, NP, HALF)
    out = pl.pallas_call(
        functools.partial(_dense_body, True),
        grid=(_DG,),
        in_specs=[
            pl.BlockSpec((NCORE, _DB, HALF), lambda i: (0, i, 0)),
            pl.BlockSpec((HALF, 2 * HALF), lambda i: (0, 0)),
            pl.BlockSpec((HALF, 2 * HALF), lambda i: (0, 0)),
            pl.BlockSpec((1, 2 * HALF), lambda i: (0, 0)),
        ],
        out_specs=pl.BlockSpec((NCORE, _DB, HALF), lambda i: (0, i, 0)),
        out_shape=jax.ShapeDtypeStruct((NCORE, NP, HALF), jnp.float32),
    )(y3, wa, wb, b)
    return out.reshape(NCORE * NP, HALF)


def _dense_final_tc(zflat, wa, wb, b):
    z3 = zflat.reshape(NCORE, NP, HALF)
    return pl.pallas_call(
        functools.partial(_dense_body, False),
        grid=(_DG,),
        in_specs=[
            pl.BlockSpec((NCORE, _DB, HALF), lambda i: (0, i, 0)),
            pl.BlockSpec((HALF, 2 * HALF), lambda i: (0, 0)),
            pl.BlockSpec((HALF, 2 * HALF), lambda i: (0, 0)),
            pl.BlockSpec((1, 2 * HALF), lambda i: (0, 0)),
        ],
        out_specs=pl.BlockSpec((_DB, 2 * HALF), lambda i: (i, 0)),
        out_shape=jax.ShapeDtypeStruct((NP, 2 * HALF), jnp.float32),
    )(z3, wa, wb, b)


def kernel(edge_index, edge_weight, emb_weight, W1, b1, W2, b2):
    pad = E_PAD - E
    srcp = jnp.pad(edge_index[1], (0, pad))
    dstp = jnp.pad(edge_index[0], (0, pad))
    wp = jnp.pad(edge_weight, (0, pad))

    xflat = jnp.zeros((NCORE * NP, HALF), jnp.float32)
    xflat = xflat.at[0:N].set(emb_weight[:, 0:HALF])
    xflat = xflat.at[NP : NP + N].set(emb_weight[:, HALF : 2 * HALF])
    zrows = jnp.zeros((NP, HALF), jnp.float32)
    b1r = b1.reshape(1, 2 * HALF)
    b2r = b2.reshape(1, 2 * HALF)

    y1 = _spmm_sc(srcp, dstp, wp, xflat, zrows)
    h = _dense_halves_tc(y1, W1[0:HALF, :], W1[HALF:, :], b1r)
    y2 = _spmm_sc(srcp, dstp, wp, h, zrows)
    z = _dense_final_tc(y2, W2[0:HALF, :], W2[HALF:, :], b2r)
    return z[0:N]


# speedup vs baseline: 9.2703x; 2.9109x over previous
import functools

import jax
import jax.numpy as jnp
from jax import lax
from jax.experimental import pallas as pl
from jax.experimental.pallas import tpu as pltpu
from jax.experimental.pallas import tpu_sc as plsc

N = 50000
E = 800000
HALF = 32
NCORE = 2
NSUB = 16
CHUNK = 128
NCHUNK = 392
EPS = CHUNK * NCHUNK
E_PAD = EPS * NSUB
STRIPE = 3128
NP = STRIPE * NSUB


BLK = 4
NBLK = NCHUNK // BLK
RB = 4


def _spmm_sc(srcp, dstp, wp, xflat, zrows):
    mesh = plsc.VectorSubcoreMesh(
        core_axis_name="c", subcore_axis_name="s", num_cores=NCORE, num_subcores=NSUB
    )

    @functools.partial(
        pl.kernel,
        out_type=jax.ShapeDtypeStruct((NCORE * NP, HALF), jnp.float32),
        mesh=mesh,
        scratch_types=[
            pltpu.VMEM((2, BLK, CHUNK), jnp.int32),
            pltpu.VMEM((2, BLK, CHUNK), jnp.int32),
            pltpu.VMEM((2, BLK, CHUNK), jnp.float32),
            pltpu.VMEM((RB, CHUNK), jnp.int32),
            pltpu.VMEM((RB, CHUNK, HALF), jnp.float32),
            pltpu.VMEM_SHARED((NP, HALF), jnp.float32),
            [pltpu.SemaphoreType.DMA] * RB,
            [pltpu.SemaphoreType.DMA] * RB,
            [pltpu.SemaphoreType.DMA] * 2,
            pltpu.SemaphoreType.DMA,
        ],
        compiler_params=pltpu.CompilerParams(use_tc_tiling_on_sc=False),
    )
    def k(src_hbm, dst_hbm, w_hbm, x_hbm, z_hbm, y_hbm,
          src2, dst2, w2, dstr, rows, acc, sem_g, sem_s, sem_i, sem_z):
        c = lax.axis_index("c")
        s = lax.axis_index("s")
        cbase = c * NP
        ebase = s * NCHUNK

        def idx_row(blk):
            return ebase + blk * BLK

        def load_idx(blk, ib, sem):
            r = idx_row(blk)
            pltpu.async_copy(src_hbm.at[pl.ds(r, BLK)], src2.at[ib], sem)
            pltpu.async_copy(dst_hbm.at[pl.ds(r, BLK)], dst2.at[ib], sem)
            pltpu.async_copy(w_hbm.at[pl.ds(r, BLK)], w2.at[ib], sem)

        def wait_idx(blk, ib, sem):
            r = idx_row(blk)
            pltpu.make_async_copy(src_hbm.at[pl.ds(r, BLK)], src2.at[ib], sem).wait()
            pltpu.make_async_copy(dst_hbm.at[pl.ds(r, BLK)], dst2.at[ib], sem).wait()
            pltpu.make_async_copy(w_hbm.at[pl.ds(r, BLK)], w2.at[ib], sem).wait()

        def bias_and_gather(ib, row, rb):
            for i in range(CHUNK // 16):
                sl = pl.ds(i * 16, 16)
                src2[ib, row, sl] = src2[ib, row, sl] + cbase
            pltpu.async_copy(x_hbm.at[src2.at[ib, row]], rows.at[rb], sem_g[rb])

        def wait_gather(ib, row, rb):
            pltpu.make_async_copy(
                x_hbm.at[src2.at[ib, row]], rows.at[rb], sem_g[rb]
            ).wait()

        def issue_scatter(ib, row, rb):
            for i in range(CHUNK // 16):
                sl = pl.ds(i * 16, 16)
                dstr[rb, sl] = dst2[ib, row, sl]
            pltpu.async_copy(rows.at[rb], acc.at[dstr.at[rb]], sem_s[rb], add=True)

        def wait_scatter(rb):
            pltpu.make_async_copy(
                rows.at[rb], acc.at[dstr.at[rb]], sem_s[rb]
            ).wait()

        def scale(ib, row, rb):
            @pl.loop(0, CHUNK // 16)
            def _(g):
                w16 = w2[ib, row, pl.ds(g * 16, 16)]
                for kk in range(16):
                    r = g * 16 + kk
                    wr = w16[kk]
                    rows[rb, r, pl.ds(0, 16)] = rows[rb, r, pl.ds(0, 16)] * wr
                    rows[rb, r, pl.ds(16, 16)] = rows[rb, r, pl.ds(16, 16)] * wr

        pltpu.async_copy(
            z_hbm.at[pl.ds(s * STRIPE, STRIPE)],
            acc.at[pl.ds(s * STRIPE, STRIPE)],
            sem_z,
        )
        load_idx(0, 0, sem_i[0])
        wait_idx(0, 0, sem_i[0])
        load_idx(1, 1, sem_i[1])
        for kk in range(RB - 1):
            bias_and_gather(0, kk, kk)
        pltpu.make_async_copy(
            z_hbm.at[pl.ds(s * STRIPE, STRIPE)],
            acc.at[pl.ds(s * STRIPE, STRIPE)],
            sem_z,
        ).wait()
        plsc.subcore_barrier()

        @pl.loop(0, NBLK // 2)
        def _(g):
            for slot in range(2):
                ib = slot
                for kk in range(BLK):
                    wait_gather(ib, kk, kk)
                    scale(ib, kk, kk)
                    if kk == 0:
                        tib, trow = ib, 3
                    else:
                        tib, trow = 1 - ib, kk - 1
                    last_block = slot == 1 and kk >= 1
                    if kk == 1:
                        def _w(g=g, slot=slot, ib=ib):
                            wait_idx(2 * g + slot + 1, 1 - ib, sem_i[1 - ib])
                        if slot == 0:
                            _w()
                        else:
                            pl.when(g < NBLK // 2 - 1)(_w)
                    def _ws(trow=trow):
                        wait_scatter(trow)
                    if kk == 0 and slot == 0:
                        pl.when(g > 0)(_ws)
                    elif last_block:
                        pl.when(g < NBLK // 2 - 1)(_ws)
                    else:
                        _ws()
                    def _ig(tib=tib, trow=trow):
                        bias_and_gather(tib, trow, trow)
                    if last_block:
                        pl.when(g < NBLK // 2 - 1)(_ig)
                    else:
                        _ig()
                    issue_scatter(ib, kk, kk)
                    if kk == 3:
                        def _li(g=g, slot=slot, ib=ib):
                            load_idx(2 * g + slot + 2, ib, sem_i[ib])
                        pl.when(g < NBLK // 2 - 1)(_li)

        for kk in range(BLK):
            wait_scatter(kk)
        plsc.subcore_barrier()
        pltpu.sync_copy(
            acc.at[pl.ds(s * STRIPE, STRIPE)],
            y_hbm.at[pl.ds(cbase + s * STRIPE, STRIPE)],
        )

    return k(srcp, dstp, wp, xflat, zrows)


_DB = 3128
_DG = NP // _DB


def _dense_body(split_out, y_ref, wa_ref, wb_ref, b_ref, o_ref):
    h = (
        jnp.dot(y_ref[0], wa_ref[...], preferred_element_type=jnp.float32,
                precision=lax.Precision.HIGHEST)
        + jnp.dot(y_ref[1], wb_ref[...], preferred_element_type=jnp.float32,
                  precision=lax.Precision.HIGHEST)
        + b_ref[...]
    )
    h = jnp.maximum(h, 0.0)
    if split_out:
        o_ref[0] = h[:, 0:HALF]
        o_ref[1] = h[:, HALF : 2 * HALF]
    else:
        o_ref[...] = h


def _dense_halves_tc(yflat, wa, wb, b):
    y3 = yflat.reshape(NCORE, NP, HALF)
    out = pl.pallas_call(
        functools.partial(_dense_body, True),
        grid=(_DG,),
        in_specs=[
            pl.BlockSpec((NCORE, _DB, HALF), lambda i: (0, i, 0)),
            pl.BlockSpec((HALF, 2 * HALF), lambda i: (0, 0)),
            pl.BlockSpec((HALF, 2 * HALF), lambda i: (0, 0)),
            pl.BlockSpec((1, 2 * HALF), lambda i: (0, 0)),
        ],
        out_specs=pl.BlockSpec((NCORE, _DB, HALF), lambda i: (0, i, 0)),
        out_shape=jax.ShapeDtypeStruct((NCORE, NP, HALF), jnp.float32),
    )(y3, wa, wb, b)
    return out.reshape(NCORE * NP, HALF)


def _dense_final_tc(zflat, wa, wb, b):
    z3 = zflat.reshape(NCORE, NP, HALF)
    return pl.pallas_call(
        functools.partial(_dense_body, False),
        grid=(_DG,),
        in_specs=[
            pl.BlockSpec((NCORE, _DB, HALF), lambda i: (0, i, 0)),
            pl.BlockSpec((HALF, 2 * HALF), lambda i: (0, 0)),
            pl.BlockSpec((HALF, 2 * HALF), lambda i: (0, 0)),
            pl.BlockSpec((1, 2 * HALF), lambda i: (0, 0)),
        ],
        out_specs=pl.BlockSpec((_DB, 2 * HALF), lambda i: (i, 0)),
        out_shape=jax.ShapeDtypeStruct((NP, 2 * HALF), jnp.float32),
    )(z3, wa, wb, b)


def kernel(edge_index, edge_weight, emb_weight, W1, b1, W2, b2):
    pad = E_PAD - E
    srcp = jnp.pad(edge_index[1], (0, pad)).reshape(E_PAD // CHUNK, CHUNK)
    dstp = jnp.pad(edge_index[0], (0, pad)).reshape(E_PAD // CHUNK, CHUNK)
    wp = jnp.pad(edge_weight, (0, pad)).reshape(E_PAD // CHUNK, CHUNK)

    xflat = jnp.zeros((NCORE * NP, HALF), jnp.float32)
    xflat = xflat.at[0:N].set(emb_weight[:, 0:HALF])
    xflat = xflat.at[NP : NP + N].set(emb_weight[:, HALF : 2 * HALF])
    zrows = jnp.zeros((NP, HALF), jnp.float32)
    b1r = b1.reshape(1, 2 * HALF)
    b2r = b2.reshape(1, 2 * HALF)

    y1 = _spmm_sc(srcp, dstp, wp, xflat, zrows)
    h = _dense_halves_tc(y1, W1[0:HALF, :], W1[HALF:, :], b1r)
    y2 = _spmm_sc(srcp, dstp, wp, h, zrows)
    z = _dense_final_tc(y2, W2[0:HALF, :], W2[HALF:, :], b2r)
    return z[0:N]


# speedup vs baseline: 11.4576x; 1.2360x over previous
import functools

import jax
import jax.numpy as jnp
from jax import lax
from jax.experimental import pallas as pl
from jax.experimental.pallas import tpu as pltpu
from jax.experimental.pallas import tpu_sc as plsc

N = 50000
E = 800000
HALF = 32
NCORE = 2
NSUB = 16
CHUNK = 128
NCHUNK = 392
EPS = CHUNK * NCHUNK
E_PAD = EPS * NSUB
STRIPE = 3128
NP = STRIPE * NSUB


BLK = 4
NBLK = NCHUNK // BLK
RB = 4


def _spmm_sc(srcp, dstp, wp, xflat, zrows):
    mesh = plsc.VectorSubcoreMesh(
        core_axis_name="c", subcore_axis_name="s", num_cores=NCORE, num_subcores=NSUB
    )

    @functools.partial(
        pl.kernel,
        out_type=jax.ShapeDtypeStruct((NCORE, NP, HALF), jnp.float32),
        mesh=mesh,
        scratch_types=[
            pltpu.VMEM((2, BLK, CHUNK), jnp.int32),
            pltpu.VMEM((2, BLK, CHUNK), jnp.int32),
            pltpu.VMEM((2, BLK, CHUNK), jnp.float32),
            pltpu.VMEM((RB, CHUNK), jnp.int32),
            pltpu.VMEM((RB, CHUNK, HALF), jnp.float32),
            pltpu.VMEM_SHARED((NP, HALF), jnp.float32),
            [pltpu.SemaphoreType.DMA] * RB,
            [pltpu.SemaphoreType.DMA] * RB,
            [pltpu.SemaphoreType.DMA] * 2,
            pltpu.SemaphoreType.DMA,
        ],
        compiler_params=pltpu.CompilerParams(use_tc_tiling_on_sc=False),
    )
    def k(src_hbm, dst_hbm, w_hbm, x_hbm, z_hbm, y_hbm,
          src2, dst2, w2, dstr, rows, acc, sem_g, sem_s, sem_i, sem_z):
        c = lax.axis_index("c")
        s = lax.axis_index("s")
        xc = x_hbm.at[c]
        yc = y_hbm.at[c]
        ebase = s * NCHUNK

        def idx_row(blk):
            return ebase + blk * BLK

        def load_idx(blk, ib, sem):
            r = idx_row(blk)
            pltpu.async_copy(src_hbm.at[pl.ds(r, BLK)], src2.at[ib], sem)
            pltpu.async_copy(dst_hbm.at[pl.ds(r, BLK)], dst2.at[ib], sem)
            pltpu.async_copy(w_hbm.at[pl.ds(r, BLK)], w2.at[ib], sem)

        def wait_idx(blk, ib, sem):
            r = idx_row(blk)
            pltpu.make_async_copy(src_hbm.at[pl.ds(r, BLK)], src2.at[ib], sem).wait()
            pltpu.make_async_copy(dst_hbm.at[pl.ds(r, BLK)], dst2.at[ib], sem).wait()
            pltpu.make_async_copy(w_hbm.at[pl.ds(r, BLK)], w2.at[ib], sem).wait()

        def bias_and_gather(ib, row, rb):
            pltpu.async_copy(xc.at[src2.at[ib, row]], rows.at[rb], sem_g[rb])

        def wait_gather(ib, row, rb):
            pltpu.make_async_copy(
                xc.at[src2.at[ib, row]], rows.at[rb], sem_g[rb]
            ).wait()

        def issue_scatter(ib, row, rb):
            for i in range(CHUNK // 16):
                sl = pl.ds(i * 16, 16)
                dstr[rb, sl] = dst2[ib, row, sl]
            pltpu.async_copy(rows.at[rb], acc.at[dstr.at[rb]], sem_s[rb], add=True)

        def wait_scatter(rb):
            pltpu.make_async_copy(
                rows.at[rb], acc.at[dstr.at[rb]], sem_s[rb]
            ).wait()

        def scale(ib, row, rb):
            @pl.loop(0, CHUNK // 16)
            def _(g):
                w16 = w2[ib, row, pl.ds(g * 16, 16)]
                for kk in range(16):
                    r = g * 16 + kk
                    wr = w16[kk]
                    rows[rb, r, pl.ds(0, 16)] = rows[rb, r, pl.ds(0, 16)] * wr
                    rows[rb, r, pl.ds(16, 16)] = rows[rb, r, pl.ds(16, 16)] * wr

        pltpu.async_copy(
            z_hbm.at[pl.ds(s * STRIPE, STRIPE)],
            acc.at[pl.ds(s * STRIPE, STRIPE)],
            sem_z,
        )
        load_idx(0, 0, sem_i[0])
        wait_idx(0, 0, sem_i[0])
        load_idx(1, 1, sem_i[1])
        for kk in range(RB - 1):
            bias_and_gather(0, kk, kk)
        pltpu.make_async_copy(
            z_hbm.at[pl.ds(s * STRIPE, STRIPE)],
            acc.at[pl.ds(s * STRIPE, STRIPE)],
            sem_z,
        ).wait()
        plsc.subcore_barrier()

        @pl.loop(0, NBLK // 2)
        def _(g):
            for slot in range(2):
                ib = slot
                for kk in range(BLK):
                    wait_gather(ib, kk, kk)
                    scale(ib, kk, kk)
                    if kk == 0:
                        tib, trow = ib, 3
                    else:
                        tib, trow = 1 - ib, kk - 1
                    last_block = slot == 1 and kk >= 1
                    if kk == 1:
                        def _w(g=g, slot=slot, ib=ib):
                            wait_idx(2 * g + slot + 1, 1 - ib, sem_i[1 - ib])
                        if slot == 0:
                            _w()
                        else:
                            pl.when(g < NBLK // 2 - 1)(_w)
                    def _ws(trow=trow):
                        wait_scatter(trow)
                    if kk == 0 and slot == 0:
                        pl.when(g > 0)(_ws)
                    elif last_block:
                        pl.when(g < NBLK // 2 - 1)(_ws)
                    else:
                        _ws()
                    def _ig(tib=tib, trow=trow):
                        bias_and_gather(tib, trow, trow)
                    if last_block:
                        pl.when(g < NBLK // 2 - 1)(_ig)
                    else:
                        _ig()
                    issue_scatter(ib, kk, kk)
                    if kk == 3:
                        def _li(g=g, slot=slot, ib=ib):
                            load_idx(2 * g + slot + 2, ib, sem_i[ib])
                        pl.when(g < NBLK // 2 - 1)(_li)

        for kk in range(BLK):
            wait_scatter(kk)
        plsc.subcore_barrier()
        pltpu.sync_copy(
            acc.at[pl.ds(s * STRIPE, STRIPE)],
            yc.at[pl.ds(s * STRIPE, STRIPE)],
        )

    return k(srcp, dstp, wp, xflat, zrows)


_DB = 4 * 3128
_DG = NP // _DB


def _dense_body(split_out, y_ref, wa_ref, wb_ref, b_ref, o_ref):
    h = (
        jnp.dot(y_ref[0], wa_ref[...], preferred_element_type=jnp.float32)
        + jnp.dot(y_ref[1], wb_ref[...], preferred_element_type=jnp.float32)
        + b_ref[...]
    )
    h = jnp.maximum(h, 0.0)
    if split_out:
        o_ref[0] = h[:, 0:HALF]
        o_ref[1] = h[:, HALF : 2 * HALF]
    else:
        o_ref[...] = h


def _dense_halves_tc(y3, wa, wb, b):
    return pl.pallas_call(
        functools.partial(_dense_body, True),
        grid=(_DG,),
        in_specs=[
            pl.BlockSpec((NCORE, _DB, HALF), lambda i: (0, i, 0)),
            pl.BlockSpec((HALF, 2 * HALF), lambda i: (0, 0)),
            pl.BlockSpec((HALF, 2 * HALF), lambda i: (0, 0)),
            pl.BlockSpec((1, 2 * HALF), lambda i: (0, 0)),
        ],
        out_specs=pl.BlockSpec((NCORE, _DB, HALF), lambda i: (0, i, 0)),
        out_shape=jax.ShapeDtypeStruct((NCORE, NP, HALF), jnp.float32),
    )(y3, wa, wb, b)


def _dense_final_tc(z3, wa, wb, b):
    return pl.pallas_call(
        functools.partial(_dense_body, False),
        grid=(_DG,),
        in_specs=[
            pl.BlockSpec((NCORE, _DB, HALF), lambda i: (0, i, 0)),
            pl.BlockSpec((HALF, 2 * HALF), lambda i: (0, 0)),
            pl.BlockSpec((HALF, 2 * HALF), lambda i: (0, 0)),
            pl.BlockSpec((1, 2 * HALF), lambda i: (0, 0)),
        ],
        out_specs=pl.BlockSpec((_DB, 2 * HALF), lambda i: (i, 0)),
        out_shape=jax.ShapeDtypeStruct((N, 2 * HALF), jnp.float32),
    )(z3, wa, wb, b)


def kernel(edge_index, edge_weight, emb_weight, W1, b1, W2, b2):
    pad = E_PAD - E
    srcp = jnp.pad(edge_index[1], (0, pad)).reshape(E_PAD // CHUNK, CHUNK)
    dstp = jnp.pad(edge_index[0], (0, pad)).reshape(E_PAD // CHUNK, CHUNK)
    wp = jnp.pad(edge_weight, (0, pad)).reshape(E_PAD // CHUNK, CHUNK)

    xh = jnp.stack([emb_weight[:, 0:HALF], emb_weight[:, HALF : 2 * HALF]])
    xh = jnp.pad(xh, ((0, 0), (0, NP - N), (0, 0)))
    zrows = jnp.zeros((NP, HALF), jnp.float32)
    b1r = b1.reshape(1, 2 * HALF)
    b2r = b2.reshape(1, 2 * HALF)

    y1 = _spmm_sc(srcp, dstp, wp, xh, zrows)
    h = _dense_halves_tc(y1, W1[0:HALF, :], W1[HALF:, :], b1r)
    y2 = _spmm_sc(srcp, dstp, wp, h, zrows)
    return _dense_final_tc(y2, W2[0:HALF, :], W2[HALF:, :], b2r)


# speedup vs baseline: 13.5303x; 1.1809x over previous
import functools

import jax
import jax.numpy as jnp
from jax import lax
from jax.experimental import pallas as pl
from jax.experimental.pallas import tpu as pltpu
from jax.experimental.pallas import tpu_sc as plsc

N = 50000
E = 800000
D = 64
HALF = 32
NCORE = 2
NSUB = 16
CHUNK = 128
NCHUNK = 392
EPS = CHUNK * NCHUNK
E_PAD = EPS * NSUB
STRIPE = 3128
NP = STRIPE * NSUB


BLK = 4
NBLK = NCHUNK // BLK
RB = 4


def _spmm_sc(srcp, dstp, wp, xflat, zrows):
    mesh = plsc.VectorSubcoreMesh(
        core_axis_name="c", subcore_axis_name="s", num_cores=NCORE, num_subcores=NSUB
    )

    @functools.partial(
        pl.kernel,
        out_type=jax.ShapeDtypeStruct((NP, NCORE, HALF), jnp.float32),
        mesh=mesh,
        scratch_types=[
            pltpu.VMEM((2, BLK, CHUNK), jnp.int32),
            pltpu.VMEM((2, BLK, CHUNK), jnp.int32),
            pltpu.VMEM((2, BLK, CHUNK), jnp.float32),
            pltpu.VMEM((RB, CHUNK), jnp.int32),
            pltpu.VMEM((RB, CHUNK, HALF), jnp.float32),
            pltpu.VMEM_SHARED((NP, HALF), jnp.float32),
            [pltpu.SemaphoreType.DMA] * RB,
            [pltpu.SemaphoreType.DMA] * RB,
            [pltpu.SemaphoreType.DMA] * 2,
            pltpu.SemaphoreType.DMA,
        ],
        compiler_params=pltpu.CompilerParams(use_tc_tiling_on_sc=False),
    )
    def k(src_hbm, dst_hbm, w_hbm, x_hbm, z_hbm, y_hbm,
          src2, dst2, w2, dstr, rows, acc, sem_g, sem_s, sem_i, sem_z):
        c = lax.axis_index("c")
        s = lax.axis_index("s")
        ebase = s * NCHUNK

        def idx_row(blk):
            return ebase + blk * BLK

        def load_idx(blk, ib, sem):
            r = idx_row(blk)
            pltpu.async_copy(src_hbm.at[pl.ds(r, BLK)], src2.at[ib], sem)
            pltpu.async_copy(dst_hbm.at[pl.ds(r, BLK)], dst2.at[ib], sem)
            pltpu.async_copy(w_hbm.at[pl.ds(r, BLK)], w2.at[ib], sem)

        def wait_idx(blk, ib, sem):
            r = idx_row(blk)
            pltpu.make_async_copy(src_hbm.at[pl.ds(r, BLK)], src2.at[ib], sem).wait()
            pltpu.make_async_copy(dst_hbm.at[pl.ds(r, BLK)], dst2.at[ib], sem).wait()
            pltpu.make_async_copy(w_hbm.at[pl.ds(r, BLK)], w2.at[ib], sem).wait()

        def bias_and_gather(ib, row, rb):
            for i in range(CHUNK // 16):
                sl = pl.ds(i * 16, 16)
                src2[ib, row, sl] = src2[ib, row, sl] + c
            pltpu.async_copy(x_hbm.at[src2.at[ib, row]], rows.at[rb], sem_g[rb])

        def wait_gather(ib, row, rb):
            pltpu.make_async_copy(
                x_hbm.at[src2.at[ib, row]], rows.at[rb], sem_g[rb]
            ).wait()

        def issue_scatter(ib, row, rb):
            for i in range(CHUNK // 16):
                sl = pl.ds(i * 16, 16)
                dstr[rb, sl] = dst2[ib, row, sl]
            pltpu.async_copy(rows.at[rb], acc.at[dstr.at[rb]], sem_s[rb], add=True)

        def wait_scatter(rb):
            pltpu.make_async_copy(
                rows.at[rb], acc.at[dstr.at[rb]], sem_s[rb]
            ).wait()

        def scale(ib, row, rb):
            @pl.loop(0, CHUNK // 16)
            def _(g):
                w16 = w2[ib, row, pl.ds(g * 16, 16)]
                for kk in range(16):
                    r = g * 16 + kk
                    wr = w16[kk]
                    rows[rb, r, pl.ds(0, 16)] = rows[rb, r, pl.ds(0, 16)] * wr
                    rows[rb, r, pl.ds(16, 16)] = rows[rb, r, pl.ds(16, 16)] * wr

        pltpu.async_copy(
            z_hbm.at[pl.ds(s * STRIPE, STRIPE)],
            acc.at[pl.ds(s * STRIPE, STRIPE)],
            sem_z,
        )
        load_idx(0, 0, sem_i[0])
        wait_idx(0, 0, sem_i[0])
        load_idx(1, 1, sem_i[1])
        for kk in range(RB - 1):
            bias_and_gather(0, kk, kk)
        pltpu.make_async_copy(
            z_hbm.at[pl.ds(s * STRIPE, STRIPE)],
            acc.at[pl.ds(s * STRIPE, STRIPE)],
            sem_z,
        ).wait()
        plsc.subcore_barrier()

        @pl.loop(0, NBLK // 2)
        def _(g):
            for slot in range(2):
                ib = slot
                for kk in range(BLK):
                    wait_gather(ib, kk, kk)
                    scale(ib, kk, kk)
                    if kk == 0:
                        tib, trow = ib, 3
                    else:
                        tib, trow = 1 - ib, kk - 1
                    last_block = slot == 1 and kk >= 1
                    if kk == 1:
                        def _w(g=g, slot=slot, ib=ib):
                            wait_idx(2 * g + slot + 1, 1 - ib, sem_i[1 - ib])
                        if slot == 0:
                            _w()
                        else:
                            pl.when(g < NBLK // 2 - 1)(_w)
                    def _ws(trow=trow):
                        wait_scatter(trow)
                    if kk == 0 and slot == 0:
                        pl.when(g > 0)(_ws)
                    elif last_block:
                        pl.when(g < NBLK // 2 - 1)(_ws)
                    else:
                        _ws()
                    def _ig(tib=tib, trow=trow):
                        bias_and_gather(tib, trow, trow)
                    if last_block:
                        pl.when(g < NBLK // 2 - 1)(_ig)
                    else:
                        _ig()
                    issue_scatter(ib, kk, kk)
                    if kk == 3:
                        def _li(g=g, slot=slot, ib=ib):
                            load_idx(2 * g + slot + 2, ib, sem_i[ib])
                        pl.when(g < NBLK // 2 - 1)(_li)

        for kk in range(BLK):
            wait_scatter(kk)
        plsc.subcore_barrier()
        pltpu.sync_copy(
            acc.at[pl.ds(s * STRIPE, STRIPE)],
            y_hbm.at[pl.ds(s * STRIPE, STRIPE), c],
        )

    return k(srcp, dstp, wp, xflat, zrows)


G2 = NP // 2
_DB = G2 // 4
_DG = 4


def _dense_body(y_ref, wd_ref, b_ref, o_ref):
    h = jnp.dot(y_ref[...], wd_ref[...], preferred_element_type=jnp.float32)
    o_ref[...] = jnp.maximum(h + b_ref[...], 0.0)


def _dense_packed_tc(yp, wd, bd):
    return pl.pallas_call(
        _dense_body,
        grid=(_DG,),
        in_specs=[
            pl.BlockSpec((_DB, 2 * D), lambda i: (i, 0)),
            pl.BlockSpec((2 * D, 2 * D), lambda i: (0, 0)),
            pl.BlockSpec((1, 2 * D), lambda i: (0, 0)),
        ],
        out_specs=pl.BlockSpec((_DB, 2 * D), lambda i: (i, 0)),
        out_shape=jax.ShapeDtypeStruct((G2, 2 * D), jnp.float32),
    )(yp, wd, bd)


def _blockdiag2(w):
    z = jnp.zeros((2 * D, 2 * D), jnp.float32)
    return z.at[0:D, 0:D].set(w).at[D : 2 * D, D : 2 * D].set(w)


def kernel(edge_index, edge_weight, emb_weight, W1, b1, W2, b2):
    pad = E_PAD - E
    srcp = jnp.pad(edge_index[1] * 2, (0, pad)).reshape(E_PAD // CHUNK, CHUNK)
    dstp = jnp.pad(edge_index[0], (0, pad)).reshape(E_PAD // CHUNK, CHUNK)
    wp = jnp.pad(edge_weight, (0, pad)).reshape(E_PAD // CHUNK, CHUNK)

    xpk = jnp.pad(emb_weight.reshape(N // 2, 2 * D), ((0, (NP - N) // 2), (0, 0)))
    zrows = jnp.zeros((NP, HALF), jnp.float32)
    w1d = _blockdiag2(W1)
    w2d = _blockdiag2(W2)
    b1d = jnp.concatenate([b1, b1]).reshape(1, 2 * D)
    b2d = jnp.concatenate([b2, b2]).reshape(1, 2 * D)

    y1 = _spmm_sc(srcp, dstp, wp, xpk.reshape(NCORE * NP, HALF), zrows)
    h = _dense_packed_tc(y1.reshape(G2, 2 * D), w1d, b1d)
    y2 = _spmm_sc(srcp, dstp, wp, h.reshape(NCORE * NP, HALF), zrows)
    zp = _dense_packed_tc(y2.reshape(G2, 2 * D), w2d, b2d)
    return zp.reshape(NP, D)[0:N]
